# baseline (device time: 16598 ns/iter reference)
import jax
import jax.numpy as jnp
from jax import lax
from jax.experimental import pallas as pl
from jax.experimental.pallas import tpu as pltpu

N_DEV = 8
M_PER = 128
H = 64
K = 1024
N_PER = 128

SX, SY, SZ, SBD, SXY, SYZ, SXZ = range(7)


def _gelu(y):
    c = 0.7978845608028654
    return 0.5 * y * (1.0 + jnp.tanh(c * (y + 0.044715 * y * y * y)))


def kernel(x, w_mat):
    def body(x_ref, w_ref, out_ref, xv_ref, wv_ref, own_ref, recv_ref,
             ov_ref, send_sems, recv_sems, load_sems, out_sems):
        my = lax.axis_index("i")

        z = my // 4
        p = my % 4
        y = p // 2
        xc = jnp.logical_or(p == 1, p == 2).astype(my.dtype)

        def pos(px, py, pz):
            return 4 * pz + 2 * py + jnp.bitwise_xor(px, py)

        xn = pos(1 - xc, y, z)
        yn = pos(xc, 1 - y, z)
        zn = pos(xc, y, 1 - z)

        slot_origin = {
            SX: xn, SY: yn, SZ: zn,
            SBD: pos(1 - xc, 1 - y, 1 - z),
            SXY: pos(1 - xc, 1 - y, z),
            SYZ: pos(xc, 1 - y, 1 - z),
            SXZ: pos(1 - xc, y, 1 - z),
        }

        x_load = pltpu.make_async_copy(x_ref, xv_ref, load_sems.at[0])
        w_load = pltpu.make_async_copy(w_ref, wv_ref, load_sems.at[1])
        x_load.start()
        w_load.start()

        barrier_sem = pltpu.get_barrier_semaphore()
        for t in (xn, yn, zn):
            pl.semaphore_signal(
                barrier_sem, inc=1,
                device_id=(t,), device_id_type=pl.DeviceIdType.MESH,
            )

        x_load.wait()
        own_ref[:, :] = xv_ref[:, :].astype(jnp.bfloat16)
        w_load.wait()
        w16 = wv_ref[:, :].astype(jnp.bfloat16)
        pl.semaphore_wait(barrier_sem, 3)

        A = pl.ds(0, H)
        B = pl.ds(H, H)

        def copy(src, dst_slot, half, sem_id, target):
            return pltpu.make_async_remote_copy(
                src_ref=src,
                dst_ref=recv_ref.at[dst_slot, half],
                send_sem=send_sems.at[sem_id],
                recv_sem=recv_sems.at[sem_id],
                device_id=(target,),
                device_id_type=pl.DeviceIdType.MESH,
            )

        p1 = [
            copy(own_ref.at[A], SX, A, 0, xn),
            copy(own_ref.at[B], SY, B, 2, yn),
            copy(own_ref.at[A], SZ, A, 4, zn),
            copy(own_ref.at[B], SX, B, 1, xn),
            copy(own_ref.at[A], SY, A, 3, yn),
            copy(own_ref.at[B], SZ, B, 5, zn),
        ]
        for s in p1:
            s.start()

        out_dmas = []

        def compute(src, origin_pos, j):
            yy = jnp.dot(src, w16, preferred_element_type=jnp.float32)
            ov_ref[j] = _gelu(yy).astype(jnp.bfloat16)
            dma = pltpu.make_async_copy(
                ov_ref.at[j],
                out_ref.at[pl.ds(origin_pos * M_PER, M_PER), :],
                out_sems.at[j],
            )
            dma.start()
            out_dmas.append(dma)

        compute(own_ref[:, :], my, 0)

        sem_dst = {
            0: (SX, A), 1: (SX, B), 2: (SY, B), 3: (SY, A),
            4: (SZ, A), 5: (SZ, B),
            6: (SYZ, A), 7: (SXY, A), 8: (SYZ, B), 9: (SXZ, A),
            10: (SXY, B), 11: (SXZ, B), 12: (SBD, A), 13: (SBD, B),
        }

        def wait(sem_id):
            slot, half = sem_dst[sem_id]
            pltpu.make_async_remote_copy(
                src_ref=recv_ref.at[slot, half],
                dst_ref=recv_ref.at[slot, half],
                send_sem=send_sems.at[sem_id],
                recv_sem=recv_sems.at[sem_id],
                device_id=(my,),
                device_id_type=pl.DeviceIdType.MESH,
            ).wait_recv()

        fwds = []

        def fwd(src_slot, src_half, dst_slot, dst_half, sem_id, target):
            r = copy(recv_ref.at[src_slot, src_half], dst_slot, dst_half,
                     sem_id, target)
            r.start()
            fwds.append(r)

        wait(4)
        fwd(SZ, A, SYZ, A, 6, yn)
        wait(0)
        fwd(SX, A, SXY, A, 7, yn)
        fwd(SX, A, SXZ, A, 9, zn)
        wait(2)
        fwd(SY, B, SYZ, B, 8, zn)
        fwd(SY, B, SXY, B, 10, xn)
        wait(5)
        fwd(SZ, B, SXZ, B, 11, xn)

        wait(6)
        fwd(SYZ, A, SBD, A, 12, xn)
        wait(10)
        fwd(SXY, B, SBD, B, 13, zn)

        wait(1)
        compute(recv_ref[SX], slot_origin[SX], 1)
        wait(3)
        compute(recv_ref[SY], slot_origin[SY], 2)
        compute(recv_ref[SZ], slot_origin[SZ], 3)

        wait(7)
        compute(recv_ref[SXY], slot_origin[SXY], 4)
        wait(8)
        compute(recv_ref[SYZ], slot_origin[SYZ], 5)
        wait(9)
        wait(11)
        compute(recv_ref[SXZ], slot_origin[SXZ], 6)

        wait(12)
        wait(13)
        compute(recv_ref[SBD], slot_origin[SBD], 7)

        for s in p1 + fwds:
            s.wait_send()
        for d in out_dmas:
            d.wait()

    return pl.pallas_call(
        body,
        out_shape=jax.ShapeDtypeStruct((N_DEV * M_PER, N_PER), jnp.bfloat16),
        in_specs=[
            pl.BlockSpec(memory_space=pl.ANY),
            pl.BlockSpec(memory_space=pl.ANY),
        ],
        out_specs=pl.BlockSpec(memory_space=pl.ANY),
        scratch_shapes=[
            pltpu.VMEM((M_PER, K), jnp.float32),
            pltpu.VMEM((K, N_PER), jnp.float32),
            pltpu.VMEM((M_PER, K), jnp.bfloat16),
            pltpu.VMEM((N_DEV - 1, M_PER, K), jnp.bfloat16),
            pltpu.VMEM((N_DEV, M_PER, N_PER), jnp.bfloat16),
            pltpu.SemaphoreType.DMA((14,)),
            pltpu.SemaphoreType.DMA((14,)),
            pltpu.SemaphoreType.DMA((2,)),
            pltpu.SemaphoreType.DMA((N_DEV,)),
        ],
        compiler_params=pltpu.CompilerParams(collective_id=0),
    )(x, w_mat)


# device time: 16475 ns/iter; 1.0075x vs baseline; 1.0075x over previous
import jax
import jax.numpy as jnp
from jax import lax
from jax.experimental import pallas as pl
from jax.experimental.pallas import tpu as pltpu

N_DEV = 8
M_PER = 128
H = 64
K = 1024
N_PER = 128

SX, SY, SZ, SBD, SXY, SYZ, SXZ = range(7)


def _gelu(y):
    c = 0.7978845608028654
    return 0.5 * y * (1.0 + jnp.tanh(c * (y + 0.044715 * y * y * y)))


def kernel(x, w_mat):
    def body(x_ref, w_ref, out_ref, own_ref, recv_ref, send_sems, recv_sems):
        my = lax.axis_index("i")

        z = my // 4
        p = my % 4
        y = p // 2
        xc = jnp.logical_or(p == 1, p == 2).astype(my.dtype)

        def pos(px, py, pz):
            return 4 * pz + 2 * py + jnp.bitwise_xor(px, py)

        xn = pos(1 - xc, y, z)
        yn = pos(xc, 1 - y, z)
        zn = pos(xc, y, 1 - z)

        slot_origin = {
            SX: xn, SY: yn, SZ: zn,
            SBD: pos(1 - xc, 1 - y, 1 - z),
            SXY: pos(1 - xc, 1 - y, z),
            SYZ: pos(xc, 1 - y, 1 - z),
            SXZ: pos(1 - xc, y, 1 - z),
        }

        barrier_sem = pltpu.get_barrier_semaphore()
        for t in (xn, yn, zn):
            pl.semaphore_signal(
                barrier_sem, inc=1,
                device_id=(t,), device_id_type=pl.DeviceIdType.MESH,
            )
        own_ref[:, :] = x_ref[:, :].astype(jnp.bfloat16)
        w16 = w_ref[:, :].astype(jnp.bfloat16)
        pl.semaphore_wait(barrier_sem, 3)

        A = pl.ds(0, H)
        B = pl.ds(H, H)

        def copy(src, dst_slot, half, sem_id, target):
            return pltpu.make_async_remote_copy(
                src_ref=src,
                dst_ref=recv_ref.at[dst_slot, half],
                send_sem=send_sems.at[sem_id],
                recv_sem=recv_sems.at[sem_id],
                device_id=(target,),
                device_id_type=pl.DeviceIdType.MESH,
            )

        p1 = [
            copy(own_ref.at[A], SX, A, 0, xn),
            copy(own_ref.at[B], SY, B, 2, yn),
            copy(own_ref.at[A], SZ, A, 4, zn),
            copy(own_ref.at[B], SX, B, 1, xn),
            copy(own_ref.at[A], SY, A, 3, yn),
            copy(own_ref.at[B], SZ, B, 5, zn),
        ]
        for s in p1:
            s.start()

        def compute(src, origin_pos):
            yy = jnp.dot(src, w16, preferred_element_type=jnp.float32)
            out_ref[pl.ds(origin_pos * M_PER, M_PER), :] = _gelu(yy)

        compute(own_ref[:, :], my)

        sem_dst = {
            0: (SX, A), 1: (SX, B), 2: (SY, B), 3: (SY, A),
            4: (SZ, A), 5: (SZ, B),
            6: (SYZ, A), 7: (SXY, A), 8: (SYZ, B), 9: (SXZ, A),
            10: (SXY, B), 11: (SXZ, B), 12: (SBD, A), 13: (SBD, B),
        }

        def wait(sem_id):
            slot, half = sem_dst[sem_id]
            pltpu.make_async_remote_copy(
                src_ref=recv_ref.at[slot, half],
                dst_ref=recv_ref.at[slot, half],
                send_sem=send_sems.at[sem_id],
                recv_sem=recv_sems.at[sem_id],
                device_id=(my,),
                device_id_type=pl.DeviceIdType.MESH,
            ).wait_recv()

        fwds = []

        def fwd(src_slot, src_half, dst_slot, dst_half, sem_id, target):
            r = copy(recv_ref.at[src_slot, src_half], dst_slot, dst_half,
                     sem_id, target)
            r.start()
            fwds.append(r)

        wait(4)
        fwd(SZ, A, SYZ, A, 6, yn)
        wait(0)
        fwd(SX, A, SXY, A, 7, yn)
        fwd(SX, A, SXZ, A, 9, zn)
        wait(2)
        fwd(SY, B, SYZ, B, 8, zn)
        fwd(SY, B, SXY, B, 10, xn)
        wait(5)
        fwd(SZ, B, SXZ, B, 11, xn)

        wait(6)
        fwd(SYZ, A, SBD, A, 12, xn)
        wait(10)
        fwd(SXY, B, SBD, B, 13, zn)

        wait(1)
        compute(recv_ref[SX], slot_origin[SX])
        wait(3)
        compute(recv_ref[SY], slot_origin[SY])
        compute(recv_ref[SZ], slot_origin[SZ])

        wait(7)
        compute(recv_ref[SXY], slot_origin[SXY])
        wait(8)
        compute(recv_ref[SYZ], slot_origin[SYZ])
        wait(9)
        wait(11)
        compute(recv_ref[SXZ], slot_origin[SXZ])

        wait(12)
        wait(13)
        compute(recv_ref[SBD], slot_origin[SBD])

        for s in p1 + fwds:
            s.wait_send()

    return pl.pallas_call(
        body,
        out_shape=jax.ShapeDtypeStruct((N_DEV * M_PER, N_PER), jnp.float32),
        in_specs=[
            pl.BlockSpec(memory_space=pltpu.VMEM),
            pl.BlockSpec(memory_space=pltpu.VMEM),
        ],
        out_specs=pl.BlockSpec(memory_space=pltpu.VMEM),
        scratch_shapes=[
            pltpu.VMEM((M_PER, K), jnp.bfloat16),
            pltpu.VMEM((N_DEV - 1, M_PER, K), jnp.bfloat16),
            pltpu.SemaphoreType.DMA((14,)),
            pltpu.SemaphoreType.DMA((14,)),
        ],
        compiler_params=pltpu.CompilerParams(collective_id=0),
    )(x, w_mat)
